# baseline (speedup 1.0000x reference)
import functools

import jax
import jax.numpy as jnp
from jax import lax
from jax.experimental import pallas as pl
from jax.experimental.pallas import tpu as pltpu
from jax.experimental.pallas import tpu_sc as plsc

_CPB = 2
_SPB = 4


def _sc_partial_sums(out3d, n_workers):
    B, G, A = out3d.shape
    w_per_b = n_workers // B
    rows_per_w = G // w_per_b
    info = plsc.get_sparse_core_info()
    NC = info.num_cores
    L = info.num_lanes
    mesh = plsc.VectorSubcoreMesh(core_axis_name="c", subcore_axis_name="s")

    @functools.partial(
        pl.kernel,
        mesh=mesh,
        out_type=jax.ShapeDtypeStruct((n_workers, A), jnp.float32),
        scratch_types=[
            pltpu.VMEM((rows_per_w, A), jnp.float32),
            pltpu.VMEM((A,), jnp.float32),
        ],
    )
    def k(in_hbm, out_hbm, buf, stg):
        wid = lax.axis_index("s") * NC + lax.axis_index("c")
        b = wid // w_per_b
        base = (wid % w_per_b) * rows_per_w
        pltpu.sync_copy(in_hbm.at[b, pl.ds(base, rows_per_w)], buf)

        nlc = A // L

        def body(r, accs):
            return tuple(accs[c] + buf[r, pl.ds(c * L, L)] for c in range(nlc))

        accs = lax.fori_loop(
            0, rows_per_w, body,
            tuple(jnp.zeros((L,), jnp.float32) for _ in range(nlc)),
            unroll=8)
        for c in range(nlc):
            stg[pl.ds(c * L, L)] = accs[c]
        pltpu.sync_copy(stg, out_hbm.at[wid])

    return k(out3d)


def _tc_body(in_hbm, ps_ref, idx_ref, w2_ref, b2_ref, w3_ref, b3_ref, o_hbm,
             obuf, sin, sout, vbuf):
    B, G, A = in_hbm.shape
    NW = ps_ref.shape[0]
    rh = G // _CPB

    def in_copy(b, j):
        return pltpu.make_async_copy(
            in_hbm.at[b, pl.ds(j * rh, rh)],
            vbuf.at[b, pl.ds(j * rh, rh)],
            sin.at[b * _CPB + j])

    for b in range(B):
        for j in range(_CPB):
            in_copy(b, j).start()

    per_b = NW // B
    iw = lax.broadcasted_iota(jnp.int32, (B, NW), 1) // per_b
    ib = lax.broadcasted_iota(jnp.int32, (B, NW), 0)
    red = (iw == ib).astype(jnp.float32)
    s = jax.lax.dot_general(
        red, ps_ref[...], (((1,), (0,)), ((), ())),
        preferred_element_type=jnp.float32)

    b2r = jnp.reshape(b2_ref[...], (1, A))
    b3r = jnp.reshape(b3_ref[...], (1, A))
    m = jax.lax.dot_general(
        s, w2_ref[...], (((1,), (0,)), ((), ())),
        preferred_element_type=jnp.float32) + b2r
    t3b = jax.lax.dot_general(
        m, w3_ref[...], (((1,), (0,)), ((), ())),
        preferred_element_type=jnp.float32) + b3r

    idxb = lax.broadcast_in_dim(idx_ref[...], (B, G), (1,))
    iota = lax.broadcasted_iota(jnp.int32, (B, G), 0)
    onehot_t = (idxb == iota).astype(jnp.float32)
    msg = jax.lax.dot_general(
        onehot_t, t3b, (((0,), (0,)), ((), ())),
        preferred_element_type=jnp.float32)

    sh = G // _SPB

    def out_copy(b, j):
        return pltpu.make_async_copy(
            obuf.at[b, pl.ds(j * sh, sh)],
            o_hbm.at[b, pl.ds(j * sh, sh)],
            sout.at[b * _SPB + j])

    for b in range(B):
        for j in range(_CPB):
            in_copy(b, j).wait()
        for j in range(_SPB):
            sl = pl.ds(j * sh, sh)
            obuf[b, sl] = jax.lax.dot_general(
                vbuf[b, sl], w3_ref[...], (((1,), (0,)), ((), ())),
                preferred_element_type=jnp.float32) + msg[j * sh:(j + 1) * sh]
            out_copy(b, j).start()
    for b in range(B):
        for j in range(_SPB):
            out_copy(b, j).wait()


def kernel(out, coords_neighbors_idx, n_batch, n_grid, n_ao, W2, b2, W3, b3):
    B, G, A = out.shape
    idx32 = coords_neighbors_idx.astype(jnp.int32)
    NW = 32
    partials = _sc_partial_sums(out, NW)
    return pl.pallas_call(
        _tc_body,
        in_specs=[
            pl.BlockSpec(memory_space=pltpu.MemorySpace.HBM),
            pl.BlockSpec(memory_space=pltpu.MemorySpace.VMEM),
            pl.BlockSpec(memory_space=pltpu.MemorySpace.VMEM),
            pl.BlockSpec(memory_space=pltpu.MemorySpace.VMEM),
            pl.BlockSpec(memory_space=pltpu.MemorySpace.VMEM),
            pl.BlockSpec(memory_space=pltpu.MemorySpace.VMEM),
            pl.BlockSpec(memory_space=pltpu.MemorySpace.VMEM),
        ],
        out_specs=pl.BlockSpec(memory_space=pltpu.MemorySpace.HBM),
        out_shape=jax.ShapeDtypeStruct((B, G, A), jnp.float32),
        scratch_shapes=[
            pltpu.VMEM((B, G, A), jnp.float32),
            pltpu.SemaphoreType.DMA((B * _CPB,)),
            pltpu.SemaphoreType.DMA((B * _SPB,)),
            pltpu.VMEM((B, G, A), jnp.float32),
        ],
    )(out, partials, idx32, W2, b2, W3, b3)


# speedup vs baseline: 2.0901x; 2.0901x over previous
import jax
import jax.numpy as jnp
from jax.experimental import pallas as pl
from jax.experimental.pallas import tpu as pltpu

_CPB = 2
_SPB = 4


def _body(in_hbm, idx_ref, w2_ref, b2_ref, w3_ref, b3_ref, o_hbm,
          vbuf, obuf, t3_s, sin, sout):
    B, G, A = in_hbm.shape
    rh = G // _CPB

    def in_copy(b, j):
        return pltpu.make_async_copy(
            in_hbm.at[b, pl.ds(j * rh, rh)],
            vbuf.at[b, pl.ds(j * rh, rh)],
            sin.at[b * _CPB + j])

    for b in range(B):
        for j in range(_CPB):
            in_copy(b, j).start()

    b2r = jnp.reshape(b2_ref[...], (1, A))
    b3r = jnp.reshape(b3_ref[...], (1, A))

    idxb = jax.lax.broadcast_in_dim(idx_ref[...], (B, G), (1,))
    iota = jax.lax.broadcasted_iota(jnp.int32, (B, G), 0)
    onehot_t = (idxb == iota).astype(jnp.float32)

    for b in range(B):
        for j in range(_CPB):
            in_copy(b, j).wait()
        s = jnp.sum(vbuf[b], axis=0, keepdims=True)
        m = jax.lax.dot_general(
            s, w2_ref[...], (((1,), (0,)), ((), ())),
            preferred_element_type=jnp.float32) + b2r
        t3_s[pl.ds(b, 1), :] = jax.lax.dot_general(
            m, w3_ref[...], (((1,), (0,)), ((), ())),
            preferred_element_type=jnp.float32) + b3r

    msg = jax.lax.dot_general(
        onehot_t, t3_s[...], (((0,), (0,)), ((), ())),
        preferred_element_type=jnp.float32)

    sh = G // _SPB

    def out_copy(b, j):
        return pltpu.make_async_copy(
            obuf.at[b, pl.ds(j * sh, sh)],
            o_hbm.at[b, pl.ds(j * sh, sh)],
            sout.at[b * _SPB + j])

    for b in range(B):
        for j in range(_SPB):
            sl = pl.ds(j * sh, sh)
            obuf[b, sl] = jax.lax.dot_general(
                vbuf[b, sl], w3_ref[...], (((1,), (0,)), ((), ())),
                preferred_element_type=jnp.float32) + msg[j * sh:(j + 1) * sh]
            out_copy(b, j).start()
    for b in range(B):
        for j in range(_SPB):
            out_copy(b, j).wait()


def kernel(out, coords_neighbors_idx, n_batch, n_grid, n_ao, W2, b2, W3, b3):
    B, G, A = out.shape
    idx32 = coords_neighbors_idx.astype(jnp.int32)
    return pl.pallas_call(
        _body,
        in_specs=[
            pl.BlockSpec(memory_space=pltpu.MemorySpace.HBM),
            pl.BlockSpec(memory_space=pltpu.MemorySpace.VMEM),
            pl.BlockSpec(memory_space=pltpu.MemorySpace.VMEM),
            pl.BlockSpec(memory_space=pltpu.MemorySpace.VMEM),
            pl.BlockSpec(memory_space=pltpu.MemorySpace.VMEM),
            pl.BlockSpec(memory_space=pltpu.MemorySpace.VMEM),
        ],
        out_specs=pl.BlockSpec(memory_space=pltpu.MemorySpace.HBM),
        out_shape=jax.ShapeDtypeStruct((B, G, A), jnp.float32),
        scratch_shapes=[
            pltpu.VMEM((B, G, A), jnp.float32),
            pltpu.VMEM((B, G, A), jnp.float32),
            pltpu.VMEM((B, A), jnp.float32),
            pltpu.SemaphoreType.DMA((B * _CPB,)),
            pltpu.SemaphoreType.DMA((B * _SPB,)),
        ],
    )(out, idx32, W2, b2, W3, b3)
